# baseline (speedup 1.0000x reference)
import functools

import jax
import jax.numpy as jnp
from jax import lax
from jax.experimental import pallas as pl
from jax.experimental.pallas import tpu as pltpu
from jax.experimental.pallas import tpu_sc as plsc

B, L, D = 16384, 50, 64
BL = B * L

CHUNK = 128
_NW = 32
N_PER_W = BL // _NW
N_CHUNKS = N_PER_W // CHUNK
K_INFLIGHT = 8
N_GROUPS = N_CHUNKS // K_INFLIGHT


def _make_sc_gather():
    info = plsc.get_sparse_core_info()
    nc, ns = info.num_cores, info.num_subcores
    assert nc * ns == _NW
    mesh = plsc.VectorSubcoreMesh(core_axis_name="c", subcore_axis_name="s")

    @functools.partial(
        pl.kernel,
        mesh=mesh,
        out_type=jax.ShapeDtypeStruct((_NW, N_CHUNKS, CHUNK), jnp.float32),
        scratch_types=[
            pltpu.VMEM((N_CHUNKS, CHUNK), jnp.int32),
            pltpu.VMEM((N_CHUNKS, CHUNK), jnp.float32),
            pltpu.SemaphoreType.DMA,
        ],
    )
    def gather_k(table_hbm, idx_hbm, out_hbm, idx_v, rows_v, sem):
        wid = lax.axis_index("s") * nc + lax.axis_index("c")
        pltpu.sync_copy(idx_hbm.at[wid], idx_v)

        def group(g, _):
            base = g * K_INFLIGHT
            descs = []
            for k in range(K_INFLIGHT):
                descs.append(
                    pltpu.async_copy(
                        table_hbm.at[idx_v.at[base + k]], rows_v.at[base + k], sem
                    )
                )
            for d in descs:
                d.wait()
            return 0

        lax.fori_loop(0, N_GROUPS, group, 0)
        pltpu.sync_copy(rows_v, out_hbm.at[wid])

    return gather_k


_sc_gather = _make_sc_gather()


BBLK = 512


def _tc_body(emb_ref, w_ref, bvec_ref, bias_ref, g_ref, eff_ref, p_ref):
    x = emb_ref[...]
    h = x + bias_ref[...]
    s = jnp.sum(h * w_ref[...], axis=-1)
    p = jax.nn.sigmoid(s + bvec_ref[0, 0])
    p_ref[...] = p
    eff_ref[...] = g_ref[...] * p


def kernel(variantxgene_embedding, variantxgene_ixs, W, b, variantxgene_effect, embedding_bias):
    ixs = variantxgene_ixs.astype(jnp.int32).reshape(_NW, N_CHUNKS, CHUNK)
    gathered = _sc_gather(variantxgene_effect, ixs).reshape(B, L)

    w3 = W.reshape(1, 1, D)
    bias3 = embedding_bias.reshape(1, 1, D)
    b2 = b.reshape(1, 1)

    grid = (B // BBLK,)
    eff, p = pl.pallas_call(
        _tc_body,
        grid=grid,
        in_specs=[
            pl.BlockSpec((BBLK, L, D), lambda i: (i, 0, 0)),
            pl.BlockSpec((1, 1, D), lambda i: (0, 0, 0)),
            pl.BlockSpec((1, 1), lambda i: (0, 0)),
            pl.BlockSpec((1, 1, D), lambda i: (0, 0, 0)),
            pl.BlockSpec((BBLK, L), lambda i: (i, 0)),
        ],
        out_specs=[
            pl.BlockSpec((BBLK, L), lambda i: (i, 0)),
            pl.BlockSpec((BBLK, L), lambda i: (i, 0)),
        ],
        out_shape=[
            jax.ShapeDtypeStruct((B, L), jnp.float32),
            jax.ShapeDtypeStruct((B, L), jnp.float32),
        ],
    )(variantxgene_embedding, w3, b2, bias3, gathered)

    return (eff, p[..., None])
